# Initial kernel scaffold; baseline (speedup 1.0000x reference)
import jax
import jax.numpy as jnp
from jax.experimental import pallas as pl


def kernel(x, W_res, g_res, b_res, W_btl, g_btl, b_btl, W_g1, g_g1, b_g1, A1, W_g2, g_g2, b_g2, A2, W_dec, g_dec, b_dec):
    raise NotImplementedError("write your pallas kernel here")


# speedup vs baseline: 2.5259x; 2.5259x over previous
import functools

import jax
import jax.numpy as jnp
from jax.experimental import pallas as pl
from jax.experimental.pallas import tpu as pltpu

B, C, N, OUT, K, HEADS = 8, 128, 2048, 128, 20, 3
C4, C2 = C // 4, C // 2
TN = 256
NT = N // TN
EPS = 1e-5
HI = jax.lax.Precision.HIGHEST


def _dot(a, b, ca, cb):
    return jax.lax.dot_general(
        a, b, (((ca,), (cb,)), ((), ())),
        preferred_element_type=jnp.float32, precision=HI)


def _leaky(x, slope=0.2):
    return jnp.where(x >= 0, x, x * slope)


def _xmom_kernel(x_ref, m2_ref, s1_ref):
    @pl.when(pl.program_id(0) == 0)
    def _():
        m2_ref[...] = jnp.zeros_like(m2_ref)
        s1_ref[...] = jnp.zeros_like(s1_ref)

    xb = x_ref[0]
    m2_ref[...] += _dot(xb, xb, 1, 1)
    s1_ref[...] += jnp.sum(xb, axis=1, keepdims=True)


def _knn_kernel(xf_ref, xt_ref, Wb_ref, bb_ref,
                G_ref, m2nn_ref, m2cr_ref, m2xx_ref, s1g_ref, s1x_ref):
    first = (pl.program_id(0) == 0) & (pl.program_id(1) == 0)

    @pl.when(first)
    def _():
        m2nn_ref[...] = jnp.zeros_like(m2nn_ref)
        m2cr_ref[...] = jnp.zeros_like(m2cr_ref)
        m2xx_ref[...] = jnp.zeros_like(m2xx_ref)
        s1g_ref[...] = jnp.zeros_like(s1g_ref)
        s1x_ref[...] = jnp.zeros_like(s1x_ref)

    xf = xf_ref[0]
    xt = xt_ref[0]
    Wb = Wb_ref[...]
    bb = bb_ref[...]

    sq = jnp.sum(xf * xf, axis=0, keepdims=True)
    Xaug = jnp.concatenate([xf * (-2.0), sq], axis=0)
    Yaug = jnp.concatenate([xt, jnp.ones((1, TN), jnp.float32)], axis=0)
    d = _dot(Xaug, Yaug, 0, 0)

    table = jnp.maximum(_dot(Wb, xf, 1, 0) + bb, 0.0)
    ctr = jnp.maximum(_dot(Wb, xt, 1, 0) + bb, 0.0)

    iota = jax.lax.broadcasted_iota(jnp.int32, (N, TN), 0)
    gsum = jnp.zeros((C4, TN), jnp.float32)
    m2nn = jnp.zeros((C4, C4), jnp.float32)
    for k in range(K):
        m = jnp.min(d, axis=0, keepdims=True)
        am = jnp.min(jnp.where(d == m, iota, N), axis=0, keepdims=True)
        sel = iota == am
        g_k = _dot(table, sel.astype(jnp.float32), 1, 0)
        G_ref[0, k] = g_k
        gsum += g_k
        m2nn += _dot(g_k, g_k, 1, 1)
        d = jnp.where(sel, jnp.inf, d)

    m2nn_ref[...] += m2nn
    m2cr_ref[...] += _dot(gsum, ctr, 1, 1)
    m2xx_ref[...] += _dot(ctr, ctr, 1, 1)
    s1g_ref[...] += jnp.sum(gsum, axis=1, keepdims=True)
    s1x_ref[...] += jnp.sum(ctr, axis=1, keepdims=True)


def _attn_branch(G_ref, hs_ref, Wa, Ap, v_base):
    S_list = []
    M = jnp.full((8, TN), -jnp.inf, jnp.float32)
    for k in range(K):
        g_k = G_ref[0, k]
        h = _leaky(_dot(Wa, g_k, 1, 0) + v_base)
        hs_ref[k] = h
        S = _leaky(_dot(Ap, h, 1, 0))
        S_list.append(S)
        M = jnp.maximum(M, S)
    den = jnp.zeros((8, TN), jnp.float32)
    e_list = []
    for k in range(K):
        e = jnp.exp(S_list[k] - M)
        e_list.append(e)
        den += e
    inv = 1.0 / den
    out = jnp.zeros((C2, TN), jnp.float32)
    for k in range(K):
        w = jnp.sum((e_list[k] * inv)[0:HEADS, :], axis=0, keepdims=True)
        out += hs_ref[k] * (w * (1.0 / HEADS))
    return out


def _gcn_kernel(G_ref, xt_ref, Wb_ref, bb_ref,
                Wa1_ref, Wb1_ref, bg1_ref, A1_ref,
                Wa2_ref, Wb2_ref, bg2_ref, A2_ref,
                A_out_ref, m2a_ref, s1a_ref,
                hs1_ref, hs2_ref):
    first = (pl.program_id(0) == 0) & (pl.program_id(1) == 0)

    @pl.when(first)
    def _():
        m2a_ref[...] = jnp.zeros_like(m2a_ref)
        s1a_ref[...] = jnp.zeros_like(s1a_ref)

    xt = xt_ref[0]
    ctr = jnp.maximum(_dot(Wb_ref[...], xt, 1, 0) + bb_ref[...], 0.0)

    v1 = _dot(Wb1_ref[...], ctr, 1, 0) + bg1_ref[...]
    out1 = _attn_branch(G_ref, hs1_ref, Wa1_ref[...], A1_ref[...], v1)
    v2 = _dot(Wb2_ref[...], ctr, 1, 0) + bg2_ref[...]
    out2 = _attn_branch(G_ref, hs2_ref, Wa2_ref[...], A2_ref[...], v2)

    maxg = G_ref[0, 0]
    for k in range(1, K):
        maxg = jnp.maximum(maxg, G_ref[0, k])

    A_tile = jnp.concatenate([out1, out2, maxg, ctr], axis=0)
    A_out_ref[0] = A_tile
    m2a_ref[...] += _dot(A_tile, A_tile, 1, 1)
    s1a_ref[...] += jnp.sum(A_tile, axis=1, keepdims=True)


def _final_kernel(A_ref, xt_ref, Wd_ref, bd_ref, Wr_ref, br_ref, o_ref):
    A_tile = A_ref[0]
    xt = xt_ref[0]
    dec = jnp.maximum(_dot(Wd_ref[...], A_tile, 1, 0) + bd_ref[...], 0.0)
    res = jnp.maximum(_dot(Wr_ref[...], xt, 1, 0) + br_ref[...], 0.0)
    o_ref[0] = dec + res


def _fold(W, g, bias, S1, M2, count):
    mu = (W @ S1[:, 0]) / count
    e2 = jnp.sum((W @ M2) * W, axis=1) / count
    var = e2 - mu * mu
    s = g / jnp.sqrt(var + EPS)
    return W * s[:, None], (bias - s * mu)[:, None]


def kernel(x, W_res, g_res, b_res, W_btl, g_btl, b_btl, W_g1, g_g1, b_g1, A1,
           W_g2, g_g2, b_g2, A2, W_dec, g_dec, b_dec):
    f32 = jnp.float32
    x = x.astype(f32)
    arb = pltpu.CompilerParams(dimension_semantics=("arbitrary", "arbitrary"))

    m2x, s1x_full = pl.pallas_call(
        _xmom_kernel,
        grid=(B,),
        in_specs=[pl.BlockSpec((1, C, N), lambda b: (b, 0, 0))],
        out_specs=[pl.BlockSpec((C, C), lambda b: (0, 0)),
                   pl.BlockSpec((C, 1), lambda b: (0, 0))],
        out_shape=[jax.ShapeDtypeStruct((C, C), f32),
                   jax.ShapeDtypeStruct((C, 1), f32)],
        compiler_params=pltpu.CompilerParams(dimension_semantics=("arbitrary",)),
    )(x)

    cnt_x = float(B * N)
    Wr, br = _fold(W_res, g_res, b_res, s1x_full, m2x, cnt_x)
    Wb, bb = _fold(W_btl, g_btl, b_btl, s1x_full, m2x, cnt_x)

    G, m2nn, m2cr, m2xx, s1g, s1c = pl.pallas_call(
        _knn_kernel,
        grid=(B, NT),
        in_specs=[pl.BlockSpec((1, C, N), lambda b, i: (b, 0, 0)),
                  pl.BlockSpec((1, C, TN), lambda b, i: (b, 0, i)),
                  pl.BlockSpec((C4, C), lambda b, i: (0, 0)),
                  pl.BlockSpec((C4, 1), lambda b, i: (0, 0))],
        out_specs=[pl.BlockSpec((1, K, C4, TN), lambda b, i: (b, 0, 0, i)),
                   pl.BlockSpec((C4, C4), lambda b, i: (0, 0)),
                   pl.BlockSpec((C4, C4), lambda b, i: (0, 0)),
                   pl.BlockSpec((C4, C4), lambda b, i: (0, 0)),
                   pl.BlockSpec((C4, 1), lambda b, i: (0, 0)),
                   pl.BlockSpec((C4, 1), lambda b, i: (0, 0))],
        out_shape=[jax.ShapeDtypeStruct((B, K, C4, N), f32),
                   jax.ShapeDtypeStruct((C4, C4), f32),
                   jax.ShapeDtypeStruct((C4, C4), f32),
                   jax.ShapeDtypeStruct((C4, C4), f32),
                   jax.ShapeDtypeStruct((C4, 1), f32),
                   jax.ShapeDtypeStruct((C4, 1), f32)],
        compiler_params=arb,
    )(x, x, Wb, bb)

    m2_feat = jnp.concatenate([
        jnp.concatenate([m2nn, m2cr], axis=1),
        jnp.concatenate([m2cr.T, float(K) * m2xx], axis=1)], axis=0)
    s1_feat = jnp.concatenate([s1g, float(K) * s1c], axis=0)
    cnt_e = float(B * N * K)
    Wg1, bg1 = _fold(W_g1, g_g1, b_g1, s1_feat, m2_feat, cnt_e)
    Wg2, bg2 = _fold(W_g2, g_g2, b_g2, s1_feat, m2_feat, cnt_e)

    A1p = jnp.zeros((8, C2), f32).at[:HEADS].set(A1)
    A2p = jnp.zeros((8, C2), f32).at[:HEADS].set(A2)

    wspec = lambda r, c: pl.BlockSpec((r, c), lambda b, i: (0, 0))
    A_feats, m2a, s1a = pl.pallas_call(
        _gcn_kernel,
        grid=(B, NT),
        in_specs=[pl.BlockSpec((1, K, C4, TN), lambda b, i: (b, 0, 0, i)),
                  pl.BlockSpec((1, C, TN), lambda b, i: (b, 0, i)),
                  wspec(C4, C), wspec(C4, 1),
                  wspec(C2, C4), wspec(C2, C4), wspec(C2, 1), wspec(8, C2),
                  wspec(C2, C4), wspec(C2, C4), wspec(C2, 1), wspec(8, C2)],
        out_specs=[pl.BlockSpec((1, 3 * C2, TN), lambda b, i: (b, 0, i)),
                   pl.BlockSpec((3 * C2, 3 * C2), lambda b, i: (0, 0)),
                   pl.BlockSpec((3 * C2, 1), lambda b, i: (0, 0))],
        out_shape=[jax.ShapeDtypeStruct((B, 3 * C2, N), f32),
                   jax.ShapeDtypeStruct((3 * C2, 3 * C2), f32),
                   jax.ShapeDtypeStruct((3 * C2, 1), f32)],
        scratch_shapes=[pltpu.VMEM((K, C2, TN), f32),
                        pltpu.VMEM((K, C2, TN), f32)],
        compiler_params=arb,
    )(G, x, Wb, bb,
      Wg1[:, :C4], Wg1[:, C4:], bg1, A1p,
      Wg2[:, :C4], Wg2[:, C4:], bg2, A2p)

    Wd, bd = _fold(W_dec, g_dec, b_dec, s1a, m2a, float(B * N))

    out = pl.pallas_call(
        _final_kernel,
        grid=(B, NT),
        in_specs=[pl.BlockSpec((1, 3 * C2, TN), lambda b, i: (b, 0, i)),
                  pl.BlockSpec((1, C, TN), lambda b, i: (b, 0, i)),
                  wspec(OUT, 3 * C2), wspec(OUT, 1),
                  wspec(OUT, C), wspec(OUT, 1)],
        out_specs=[pl.BlockSpec((1, OUT, TN), lambda b, i: (b, 0, i))],
        out_shape=[jax.ShapeDtypeStruct((B, OUT, N), f32)],
        compiler_params=arb,
    )(A_feats, x, Wd, bd, Wr, br)[0]

    return out[..., None]


# speedup vs baseline: 6.5640x; 2.5987x over previous
import functools

import jax
import jax.numpy as jnp
from jax.experimental import pallas as pl
from jax.experimental.pallas import tpu as pltpu

B, C, N, OUT, K, HEADS = 8, 128, 2048, 128, 20, 3
C4, C2 = C // 4, C // 2
TN = 256
NT = N // TN
EPS = 1e-5
HI = jax.lax.Precision.HIGHEST


def _dot(a, b, ca, cb):
    return jax.lax.dot_general(
        a, b, (((ca,), (cb,)), ((), ())),
        preferred_element_type=jnp.float32, precision=HI)


def _leaky(x, slope=0.2):
    return jnp.where(x >= 0, x, x * slope)


def _xmom_kernel(x_ref, m2_ref, s1_ref):
    @pl.when(pl.program_id(0) == 0)
    def _():
        m2_ref[...] = jnp.zeros_like(m2_ref)
        s1_ref[...] = jnp.zeros_like(s1_ref)

    xb = x_ref[0]
    m2_ref[...] += _dot(xb, xb, 1, 1)
    s1_ref[...] += jnp.sum(xb, axis=1, keepdims=True)


def _knn_kernel(xf_ref, xt_ref, Wb_ref, bb_ref,
                G_ref, m2nn_ref, m2cr_ref, m2xx_ref, s1g_ref, s1x_ref):
    first = (pl.program_id(0) == 0) & (pl.program_id(1) == 0)

    @pl.when(first)
    def _():
        m2nn_ref[...] = jnp.zeros_like(m2nn_ref)
        m2cr_ref[...] = jnp.zeros_like(m2cr_ref)
        m2xx_ref[...] = jnp.zeros_like(m2xx_ref)
        s1g_ref[...] = jnp.zeros_like(s1g_ref)
        s1x_ref[...] = jnp.zeros_like(s1x_ref)

    xf = xf_ref[0]
    xt = xt_ref[0]
    Wb = Wb_ref[...]
    bb = bb_ref[...]

    sq = jnp.sum(xf * xf, axis=0, keepdims=True)
    Xaug = jnp.concatenate([xf * (-2.0), sq], axis=0)
    Yaug = jnp.concatenate([xt, jnp.ones((1, TN), jnp.float32)], axis=0)
    d = _dot(Xaug, Yaug, 0, 0)

    table = jnp.maximum(_dot(Wb, xf, 1, 0) + bb, 0.0)
    ctr = jnp.maximum(_dot(Wb, xt, 1, 0) + bb, 0.0)

    t_hi = table.astype(jnp.bfloat16)
    t_lo = (table - t_hi.astype(jnp.float32)).astype(jnp.bfloat16)

    iota = jax.lax.broadcasted_iota(jnp.int32, (N, TN), 0)
    gs = []
    for k in range(K):
        m = jnp.min(d, axis=0, keepdims=True)
        am = jnp.min(jnp.where(d == m, iota, N), axis=0, keepdims=True)
        sel = iota == am
        sel_bf = jnp.where(sel, 1.0, 0.0).astype(jnp.bfloat16)
        g_k = (jax.lax.dot_general(t_hi, sel_bf, (((1,), (0,)), ((), ())),
                                   preferred_element_type=jnp.float32)
               + jax.lax.dot_general(t_lo, sel_bf, (((1,), (0,)), ((), ())),
                                     preferred_element_type=jnp.float32))
        G_ref[0, k] = g_k
        gs.append(g_k)
        d = jnp.where(sel, jnp.inf, d)

    gsum = gs[0]
    for k in range(1, K):
        gsum = gsum + gs[k]
    gcat = jnp.concatenate(gs, axis=1)
    m2nn_ref[...] += _dot(gcat, gcat, 1, 1)
    m2cr_ref[...] += _dot(gsum, ctr, 1, 1)
    m2xx_ref[...] += _dot(ctr, ctr, 1, 1)
    s1g_ref[...] += jnp.sum(gsum, axis=1, keepdims=True)
    s1x_ref[...] += jnp.sum(ctr, axis=1, keepdims=True)


def _attn_branch(gbig, Wa, Ap, v_base):
    vcat = jnp.concatenate([v_base] * K, axis=1)
    hbig = _leaky(_dot(Wa, gbig, 1, 0) + vcat)
    sbig = _leaky(_dot(Ap, hbig, 1, 0))
    s_ks = [sbig[:, k * TN:(k + 1) * TN] for k in range(K)]
    M = s_ks[0]
    for k in range(1, K):
        M = jnp.maximum(M, s_ks[k])
    e_list = [jnp.exp(s - M) for s in s_ks]
    den = e_list[0]
    for k in range(1, K):
        den = den + e_list[k]
    inv = 1.0 / den
    out = jnp.zeros((C2, TN), jnp.float32)
    for k in range(K):
        w = jnp.sum((e_list[k] * inv)[0:HEADS, :], axis=0, keepdims=True)
        out += hbig[:, k * TN:(k + 1) * TN] * (w * (1.0 / HEADS))
    return out


def _gcn_kernel(G_ref, xt_ref, Wb_ref, bb_ref,
                Wa1_ref, Wb1_ref, bg1_ref, A1_ref,
                Wa2_ref, Wb2_ref, bg2_ref, A2_ref,
                A_out_ref, m2a_ref, s1a_ref):
    first = (pl.program_id(0) == 0) & (pl.program_id(1) == 0)

    @pl.when(first)
    def _():
        m2a_ref[...] = jnp.zeros_like(m2a_ref)
        s1a_ref[...] = jnp.zeros_like(s1a_ref)

    xt = xt_ref[0]
    ctr = jnp.maximum(_dot(Wb_ref[...], xt, 1, 0) + bb_ref[...], 0.0)

    g_ks = [G_ref[0, k] for k in range(K)]
    gbig = jnp.concatenate(g_ks, axis=1)

    v1 = _dot(Wb1_ref[...], ctr, 1, 0) + bg1_ref[...]
    out1 = _attn_branch(gbig, Wa1_ref[...], A1_ref[...], v1)
    v2 = _dot(Wb2_ref[...], ctr, 1, 0) + bg2_ref[...]
    out2 = _attn_branch(gbig, Wa2_ref[...], A2_ref[...], v2)

    maxg = g_ks[0]
    for k in range(1, K):
        maxg = jnp.maximum(maxg, g_ks[k])

    A_tile = jnp.concatenate([out1, out2, maxg, ctr], axis=0)
    A_out_ref[0] = A_tile
    m2a_ref[...] += _dot(A_tile, A_tile, 1, 1)
    s1a_ref[...] += jnp.sum(A_tile, axis=1, keepdims=True)


def _final_kernel(A_ref, xt_ref, Wd_ref, bd_ref, Wr_ref, br_ref, o_ref):
    A_tile = A_ref[0]
    xt = xt_ref[0]
    dec = jnp.maximum(_dot(Wd_ref[...], A_tile, 1, 0) + bd_ref[...], 0.0)
    res = jnp.maximum(_dot(Wr_ref[...], xt, 1, 0) + br_ref[...], 0.0)
    o_ref[0] = dec + res


def _fold(W, g, bias, S1, M2, count):
    mu = (W @ S1[:, 0]) / count
    e2 = jnp.sum((W @ M2) * W, axis=1) / count
    var = e2 - mu * mu
    s = g / jnp.sqrt(var + EPS)
    return W * s[:, None], (bias - s * mu)[:, None]


def kernel(x, W_res, g_res, b_res, W_btl, g_btl, b_btl, W_g1, g_g1, b_g1, A1,
           W_g2, g_g2, b_g2, A2, W_dec, g_dec, b_dec):
    f32 = jnp.float32
    x = x.astype(f32)
    arb = pltpu.CompilerParams(dimension_semantics=("arbitrary", "arbitrary"))

    m2x, s1x_full = pl.pallas_call(
        _xmom_kernel,
        grid=(B,),
        in_specs=[pl.BlockSpec((1, C, N), lambda b: (b, 0, 0))],
        out_specs=[pl.BlockSpec((C, C), lambda b: (0, 0)),
                   pl.BlockSpec((C, 1), lambda b: (0, 0))],
        out_shape=[jax.ShapeDtypeStruct((C, C), f32),
                   jax.ShapeDtypeStruct((C, 1), f32)],
        compiler_params=pltpu.CompilerParams(dimension_semantics=("arbitrary",)),
    )(x)

    cnt_x = float(B * N)
    Wr, br = _fold(W_res, g_res, b_res, s1x_full, m2x, cnt_x)
    Wb, bb = _fold(W_btl, g_btl, b_btl, s1x_full, m2x, cnt_x)

    G, m2nn, m2cr, m2xx, s1g, s1c = pl.pallas_call(
        _knn_kernel,
        grid=(B, NT),
        in_specs=[pl.BlockSpec((1, C, N), lambda b, i: (b, 0, 0)),
                  pl.BlockSpec((1, C, TN), lambda b, i: (b, 0, i)),
                  pl.BlockSpec((C4, C), lambda b, i: (0, 0)),
                  pl.BlockSpec((C4, 1), lambda b, i: (0, 0))],
        out_specs=[pl.BlockSpec((1, K, C4, TN), lambda b, i: (b, 0, 0, i)),
                   pl.BlockSpec((C4, C4), lambda b, i: (0, 0)),
                   pl.BlockSpec((C4, C4), lambda b, i: (0, 0)),
                   pl.BlockSpec((C4, C4), lambda b, i: (0, 0)),
                   pl.BlockSpec((C4, 1), lambda b, i: (0, 0)),
                   pl.BlockSpec((C4, 1), lambda b, i: (0, 0))],
        out_shape=[jax.ShapeDtypeStruct((B, K, C4, N), f32),
                   jax.ShapeDtypeStruct((C4, C4), f32),
                   jax.ShapeDtypeStruct((C4, C4), f32),
                   jax.ShapeDtypeStruct((C4, C4), f32),
                   jax.ShapeDtypeStruct((C4, 1), f32),
                   jax.ShapeDtypeStruct((C4, 1), f32)],
        compiler_params=arb,
    )(x, x, Wb, bb)

    m2_feat = jnp.concatenate([
        jnp.concatenate([m2nn, m2cr], axis=1),
        jnp.concatenate([m2cr.T, float(K) * m2xx], axis=1)], axis=0)
    s1_feat = jnp.concatenate([s1g, float(K) * s1c], axis=0)
    cnt_e = float(B * N * K)
    Wg1, bg1 = _fold(W_g1, g_g1, b_g1, s1_feat, m2_feat, cnt_e)
    Wg2, bg2 = _fold(W_g2, g_g2, b_g2, s1_feat, m2_feat, cnt_e)

    A1p = jnp.zeros((8, C2), f32).at[:HEADS].set(A1)
    A2p = jnp.zeros((8, C2), f32).at[:HEADS].set(A2)

    wspec = lambda r, c: pl.BlockSpec((r, c), lambda b, i: (0, 0))
    A_feats, m2a, s1a = pl.pallas_call(
        _gcn_kernel,
        grid=(B, NT),
        in_specs=[pl.BlockSpec((1, K, C4, TN), lambda b, i: (b, 0, 0, i)),
                  pl.BlockSpec((1, C, TN), lambda b, i: (b, 0, i)),
                  wspec(C4, C), wspec(C4, 1),
                  wspec(C2, C4), wspec(C2, C4), wspec(C2, 1), wspec(8, C2),
                  wspec(C2, C4), wspec(C2, C4), wspec(C2, 1), wspec(8, C2)],
        out_specs=[pl.BlockSpec((1, 3 * C2, TN), lambda b, i: (b, 0, i)),
                   pl.BlockSpec((3 * C2, 3 * C2), lambda b, i: (0, 0)),
                   pl.BlockSpec((3 * C2, 1), lambda b, i: (0, 0))],
        out_shape=[jax.ShapeDtypeStruct((B, 3 * C2, N), f32),
                   jax.ShapeDtypeStruct((3 * C2, 3 * C2), f32),
                   jax.ShapeDtypeStruct((3 * C2, 1), f32)],
        compiler_params=arb,
    )(G, x, Wb, bb,
      Wg1[:, :C4], Wg1[:, C4:], bg1, A1p,
      Wg2[:, :C4], Wg2[:, C4:], bg2, A2p)

    Wd, bd = _fold(W_dec, g_dec, b_dec, s1a, m2a, float(B * N))

    out = pl.pallas_call(
        _final_kernel,
        grid=(B, NT),
        in_specs=[pl.BlockSpec((1, 3 * C2, TN), lambda b, i: (b, 0, i)),
                  pl.BlockSpec((1, C, TN), lambda b, i: (b, 0, i)),
                  wspec(OUT, 3 * C2), wspec(OUT, 1),
                  wspec(OUT, C), wspec(OUT, 1)],
        out_specs=[pl.BlockSpec((1, OUT, TN), lambda b, i: (b, 0, i))],
        out_shape=[jax.ShapeDtypeStruct((B, OUT, N), f32)],
        compiler_params=arb,
    )(A_feats, x, Wd, bd, Wr, br)[0]

    return out[..., None]


# speedup vs baseline: 7.6745x; 1.1692x over previous
import functools

import jax
import jax.numpy as jnp
from jax.experimental import pallas as pl
from jax.experimental.pallas import tpu as pltpu

B, C, N, OUT, K, HEADS = 8, 128, 2048, 128, 20, 3
C4, C2 = C // 4, C // 2
TN = 256
NT = N // TN
EPS = 1e-5
HI = jax.lax.Precision.HIGHEST


def _dot(a, b, ca, cb):
    return jax.lax.dot_general(
        a, b, (((ca,), (cb,)), ((), ())),
        preferred_element_type=jnp.float32, precision=HI)


def _leaky(x, slope=0.2):
    return jnp.where(x >= 0, x, x * slope)


def _xmom_kernel(x_ref, m2_ref, s1_ref):
    @pl.when(pl.program_id(0) == 0)
    def _():
        m2_ref[...] = jnp.zeros_like(m2_ref)
        s1_ref[...] = jnp.zeros_like(s1_ref)

    xb = x_ref[0]
    m2_ref[...] += _dot(xb, xb, 1, 1)
    s1_ref[...] += jnp.sum(xb, axis=1, keepdims=True)


def _knn_kernel(xf_ref, xt_ref, Wb_ref, bb_ref,
                G_ref, m2nn_ref, m2cr_ref, m2xx_ref, s1g_ref, s1x_ref):
    first = (pl.program_id(0) == 0) & (pl.program_id(1) == 0)

    @pl.when(first)
    def _():
        m2nn_ref[...] = jnp.zeros_like(m2nn_ref)
        m2cr_ref[...] = jnp.zeros_like(m2cr_ref)
        m2xx_ref[...] = jnp.zeros_like(m2xx_ref)
        s1g_ref[...] = jnp.zeros_like(s1g_ref)
        s1x_ref[...] = jnp.zeros_like(s1x_ref)

    xf = xf_ref[0]
    xt = xt_ref[0]
    Wb = Wb_ref[...]
    bb = bb_ref[...]

    sq = jnp.sum(xf * xf, axis=0, keepdims=True)
    Xaug = jnp.concatenate([xf * (-2.0), sq], axis=0)
    Yaug = jnp.concatenate([xt, jnp.ones((1, TN), jnp.float32)], axis=0)
    d = _dot(Xaug, Yaug, 0, 0)

    table = jnp.maximum(_dot(Wb, xf, 1, 0) + bb, 0.0)
    ctr = jnp.maximum(_dot(Wb, xt, 1, 0) + bb, 0.0)

    t_hi = table.astype(jnp.bfloat16)
    t_lo = (table - t_hi.astype(jnp.float32)).astype(jnp.bfloat16)
    t_hl = jnp.concatenate([t_hi, t_lo], axis=0)

    iota = jax.lax.broadcasted_iota(jnp.int32, (N, TN), 0).astype(jnp.float32)
    gs = []
    for k in range(K):
        m = jnp.min(d, axis=0, keepdims=True)
        am = jnp.min(jnp.where(d == m, iota, float(N)), axis=0, keepdims=True)
        sel = iota == am
        sel_bf = jnp.where(sel, 1.0, 0.0).astype(jnp.bfloat16)
        ghl = jax.lax.dot_general(t_hl, sel_bf, (((1,), (0,)), ((), ())),
                                  preferred_element_type=jnp.float32)
        g_k = ghl[:C4] + ghl[C4:]
        G_ref[0, k] = g_k
        gs.append(g_k)
        d = jnp.where(sel, jnp.inf, d)

    gsum = gs[0]
    for k in range(1, K):
        gsum = gsum + gs[k]
    gcat = jnp.concatenate(gs, axis=1)
    m2nn_ref[...] += _dot(gcat, gcat, 1, 1)
    m2cr_ref[...] += _dot(gsum, ctr, 1, 1)
    m2xx_ref[...] += _dot(ctr, ctr, 1, 1)
    s1g_ref[...] += jnp.sum(gsum, axis=1, keepdims=True)
    s1x_ref[...] += jnp.sum(ctr, axis=1, keepdims=True)


def _attn_branch(hbig, Ap):
    sbig = _leaky(_dot(Ap, hbig, 1, 0))
    s_ks = [sbig[:, k * TN:(k + 1) * TN] for k in range(K)]
    M = s_ks[0]
    for k in range(1, K):
        M = jnp.maximum(M, s_ks[k])
    e_list = [jnp.exp(s - M) for s in s_ks]
    den = e_list[0]
    for k in range(1, K):
        den = den + e_list[k]
    inv = 1.0 / den
    out = jnp.zeros((C2, TN), jnp.float32)
    for k in range(K):
        w = jnp.sum((e_list[k] * inv)[0:HEADS, :], axis=0, keepdims=True)
        out += hbig[:, k * TN:(k + 1) * TN] * (w * (1.0 / HEADS))
    return out


def _gcn_kernel(G_ref, xt_ref, Wb_ref, bb_ref,
                Wa12_ref, Wb12_ref, bg12_ref, A1_ref, A2_ref,
                A_out_ref, m2a_ref, s1a_ref):
    first = (pl.program_id(0) == 0) & (pl.program_id(1) == 0)

    @pl.when(first)
    def _():
        m2a_ref[...] = jnp.zeros_like(m2a_ref)
        s1a_ref[...] = jnp.zeros_like(s1a_ref)

    xt = xt_ref[0]
    ctr = jnp.maximum(_dot(Wb_ref[...], xt, 1, 0) + bb_ref[...], 0.0)

    g_ks = [G_ref[0, k] for k in range(K)]
    gbig = jnp.concatenate(g_ks, axis=1)

    ubig = _dot(Wa12_ref[...], gbig, 1, 0)
    v12 = _dot(Wb12_ref[...], ctr, 1, 0) + bg12_ref[...]
    vcat = jnp.concatenate([v12] * K, axis=1)
    hbig = _leaky(ubig + vcat)
    out1 = _attn_branch(hbig[:C2], A1_ref[...])
    out2 = _attn_branch(hbig[C2:], A2_ref[...])

    maxg = g_ks[0]
    for k in range(1, K):
        maxg = jnp.maximum(maxg, g_ks[k])

    A_tile = jnp.concatenate([out1, out2, maxg, ctr], axis=0)
    A_out_ref[0] = A_tile
    m2a_ref[...] += _dot(A_tile, A_tile, 1, 1)
    s1a_ref[...] += jnp.sum(A_tile, axis=1, keepdims=True)


def _final_kernel(A_ref, xt_ref, Wd_ref, bd_ref, Wr_ref, br_ref, o_ref):
    A_tile = A_ref[0]
    xt = xt_ref[0]
    dec = jnp.maximum(_dot(Wd_ref[...], A_tile, 1, 0) + bd_ref[...], 0.0)
    res = jnp.maximum(_dot(Wr_ref[...], xt, 1, 0) + br_ref[...], 0.0)
    o_ref[0] = dec + res


def _fold(W, g, bias, S1, M2, count):
    mu = (W @ S1[:, 0]) / count
    e2 = jnp.sum((W @ M2) * W, axis=1) / count
    var = e2 - mu * mu
    s = g / jnp.sqrt(var + EPS)
    return W * s[:, None], (bias - s * mu)[:, None]


def kernel(x, W_res, g_res, b_res, W_btl, g_btl, b_btl, W_g1, g_g1, b_g1, A1,
           W_g2, g_g2, b_g2, A2, W_dec, g_dec, b_dec):
    f32 = jnp.float32
    x = x.astype(f32)
    arb = pltpu.CompilerParams(dimension_semantics=("arbitrary", "arbitrary"))

    m2x, s1x_full = pl.pallas_call(
        _xmom_kernel,
        grid=(B,),
        in_specs=[pl.BlockSpec((1, C, N), lambda b: (b, 0, 0))],
        out_specs=[pl.BlockSpec((C, C), lambda b: (0, 0)),
                   pl.BlockSpec((C, 1), lambda b: (0, 0))],
        out_shape=[jax.ShapeDtypeStruct((C, C), f32),
                   jax.ShapeDtypeStruct((C, 1), f32)],
        compiler_params=pltpu.CompilerParams(dimension_semantics=("arbitrary",)),
    )(x)

    cnt_x = float(B * N)
    Wr, br = _fold(W_res, g_res, b_res, s1x_full, m2x, cnt_x)
    Wb, bb = _fold(W_btl, g_btl, b_btl, s1x_full, m2x, cnt_x)

    G, m2nn, m2cr, m2xx, s1g, s1c = pl.pallas_call(
        _knn_kernel,
        grid=(B, NT),
        in_specs=[pl.BlockSpec((1, C, N), lambda b, i: (b, 0, 0)),
                  pl.BlockSpec((1, C, TN), lambda b, i: (b, 0, i)),
                  pl.BlockSpec((C4, C), lambda b, i: (0, 0)),
                  pl.BlockSpec((C4, 1), lambda b, i: (0, 0))],
        out_specs=[pl.BlockSpec((1, K, C4, TN), lambda b, i: (b, 0, 0, i)),
                   pl.BlockSpec((C4, C4), lambda b, i: (0, 0)),
                   pl.BlockSpec((C4, C4), lambda b, i: (0, 0)),
                   pl.BlockSpec((C4, C4), lambda b, i: (0, 0)),
                   pl.BlockSpec((C4, 1), lambda b, i: (0, 0)),
                   pl.BlockSpec((C4, 1), lambda b, i: (0, 0))],
        out_shape=[jax.ShapeDtypeStruct((B, K, C4, N), f32),
                   jax.ShapeDtypeStruct((C4, C4), f32),
                   jax.ShapeDtypeStruct((C4, C4), f32),
                   jax.ShapeDtypeStruct((C4, C4), f32),
                   jax.ShapeDtypeStruct((C4, 1), f32),
                   jax.ShapeDtypeStruct((C4, 1), f32)],
        compiler_params=arb,
    )(x, x, Wb, bb)

    m2_feat = jnp.concatenate([
        jnp.concatenate([m2nn, m2cr], axis=1),
        jnp.concatenate([m2cr.T, float(K) * m2xx], axis=1)], axis=0)
    s1_feat = jnp.concatenate([s1g, float(K) * s1c], axis=0)
    cnt_e = float(B * N * K)
    Wg1, bg1 = _fold(W_g1, g_g1, b_g1, s1_feat, m2_feat, cnt_e)
    Wg2, bg2 = _fold(W_g2, g_g2, b_g2, s1_feat, m2_feat, cnt_e)

    A1p = jnp.zeros((8, C2), f32).at[:HEADS].set(A1)
    A2p = jnp.zeros((8, C2), f32).at[:HEADS].set(A2)

    wspec = lambda r, c: pl.BlockSpec((r, c), lambda b, i: (0, 0))
    A_feats, m2a, s1a = pl.pallas_call(
        _gcn_kernel,
        grid=(B, NT),
        in_specs=[pl.BlockSpec((1, K, C4, TN), lambda b, i: (b, 0, 0, i)),
                  pl.BlockSpec((1, C, TN), lambda b, i: (b, 0, i)),
                  wspec(C4, C), wspec(C4, 1),
                  wspec(2 * C2, C4), wspec(2 * C2, C4), wspec(2 * C2, 1),
                  wspec(8, C2), wspec(8, C2)],
        out_specs=[pl.BlockSpec((1, 3 * C2, TN), lambda b, i: (b, 0, i)),
                   pl.BlockSpec((3 * C2, 3 * C2), lambda b, i: (0, 0)),
                   pl.BlockSpec((3 * C2, 1), lambda b, i: (0, 0))],
        out_shape=[jax.ShapeDtypeStruct((B, 3 * C2, N), f32),
                   jax.ShapeDtypeStruct((3 * C2, 3 * C2), f32),
                   jax.ShapeDtypeStruct((3 * C2, 1), f32)],
        compiler_params=arb,
    )(G, x, Wb, bb,
      jnp.concatenate([Wg1[:, :C4], Wg2[:, :C4]], axis=0),
      jnp.concatenate([Wg1[:, C4:], Wg2[:, C4:]], axis=0),
      jnp.concatenate([bg1, bg2], axis=0), A1p, A2p)

    Wd, bd = _fold(W_dec, g_dec, b_dec, s1a, m2a, float(B * N))

    out = pl.pallas_call(
        _final_kernel,
        grid=(B, NT),
        in_specs=[pl.BlockSpec((1, 3 * C2, TN), lambda b, i: (b, 0, i)),
                  pl.BlockSpec((1, C, TN), lambda b, i: (b, 0, i)),
                  wspec(OUT, 3 * C2), wspec(OUT, 1),
                  wspec(OUT, C), wspec(OUT, 1)],
        out_specs=[pl.BlockSpec((1, OUT, TN), lambda b, i: (b, 0, i))],
        out_shape=[jax.ShapeDtypeStruct((B, OUT, N), f32)],
        compiler_params=arb,
    )(A_feats, x, Wd, bd, Wr, br)[0]

    return out[..., None]


# speedup vs baseline: 10.9132x; 1.4220x over previous
import functools

import jax
import jax.numpy as jnp
from jax.experimental import pallas as pl
from jax.experimental.pallas import tpu as pltpu

B, C, N, OUT, K, HEADS = 8, 128, 2048, 128, 20, 3
C4, C2 = C // 4, C // 2
TN = 256
NT = N // TN
EPS = 1e-5
HI = jax.lax.Precision.HIGHEST


def _dot(a, b, ca, cb):
    return jax.lax.dot_general(
        a, b, (((ca,), (cb,)), ((), ())),
        preferred_element_type=jnp.float32, precision=HI)


def _leaky(x, slope=0.2):
    return jnp.where(x >= 0, x, x * slope)


def _xmom_kernel(x_ref, m2_ref, s1_ref):
    @pl.when(pl.program_id(0) == 0)
    def _():
        m2_ref[...] = jnp.zeros_like(m2_ref)
        s1_ref[...] = jnp.zeros_like(s1_ref)

    xb = x_ref[0]
    m2_ref[...] += _dot(xb, xb, 1, 1)
    s1_ref[...] += jnp.sum(xb, axis=1, keepdims=True)


def _knn_kernel(xf_ref, xt_ref, Wb_ref, bb_ref,
                G_ref, m2nn_ref, m2cr_ref, m2xx_ref, s1g_ref, s1x_ref):
    first = (pl.program_id(0) == 0) & (pl.program_id(1) == 0)

    @pl.when(first)
    def _():
        m2nn_ref[...] = jnp.zeros_like(m2nn_ref)
        m2cr_ref[...] = jnp.zeros_like(m2cr_ref)
        m2xx_ref[...] = jnp.zeros_like(m2xx_ref)
        s1g_ref[...] = jnp.zeros_like(s1g_ref)
        s1x_ref[...] = jnp.zeros_like(s1x_ref)

    xf = xf_ref[0]
    xt = xt_ref[0]
    Wb = Wb_ref[...]
    bb = bb_ref[...]

    sq = jnp.sum(xf * xf, axis=0, keepdims=True)
    Xaug = jnp.concatenate([xf * (-2.0), sq], axis=0)
    Yaug = jnp.concatenate([xt, jnp.ones((1, TN), jnp.float32)], axis=0)
    Xh = Xaug.astype(jnp.bfloat16)
    Xl = (Xaug - Xh.astype(jnp.float32)).astype(jnp.bfloat16)
    Yh = Yaug.astype(jnp.bfloat16)
    Yl = (Yaug - Yh.astype(jnp.float32)).astype(jnp.bfloat16)
    X3 = jnp.concatenate([Xh, Xl, Xh], axis=0)
    Y3 = jnp.concatenate([Yh, Yh, Yl], axis=0)
    d = jax.lax.dot_general(X3, Y3, (((0,), (0,)), ((), ())),
                            preferred_element_type=jnp.float32)

    table = jnp.maximum(_dot(Wb, xf, 1, 0) + bb, 0.0)
    ctr = jnp.maximum(_dot(Wb, xt, 1, 0) + bb, 0.0)

    t_hi = table.astype(jnp.bfloat16)
    t_lo = (table - t_hi.astype(jnp.float32)).astype(jnp.bfloat16)
    t_hl = jnp.concatenate([t_hi, t_lo], axis=0)

    iota = jax.lax.broadcasted_iota(jnp.int32, (N, TN), 0)
    gs = []
    for k in range(K):
        am = jnp.argmin(d, axis=0)
        sel = iota == am[None, :]
        sel_bf = jnp.where(sel, 1.0, 0.0).astype(jnp.bfloat16)
        ghl = jax.lax.dot_general(t_hl, sel_bf, (((1,), (0,)), ((), ())),
                                  preferred_element_type=jnp.float32)
        g_k = ghl[:C4] + ghl[C4:]
        G_ref[0, k] = g_k
        gs.append(g_k)
        d = jnp.where(sel, jnp.inf, d)

    gsum = gs[0]
    for k in range(1, K):
        gsum = gsum + gs[k]
    gcat = jnp.concatenate(gs, axis=1)
    m2nn_ref[...] += _dot(gcat, gcat, 1, 1)
    m2cr_ref[...] += _dot(gsum, ctr, 1, 1)
    m2xx_ref[...] += _dot(ctr, ctr, 1, 1)
    s1g_ref[...] += jnp.sum(gsum, axis=1, keepdims=True)
    s1x_ref[...] += jnp.sum(ctr, axis=1, keepdims=True)


def _attn_branch(hbig, Ap):
    sbig = _leaky(_dot(Ap, hbig, 1, 0))
    s_ks = [sbig[:, k * TN:(k + 1) * TN] for k in range(K)]
    M = s_ks[0]
    for k in range(1, K):
        M = jnp.maximum(M, s_ks[k])
    e_list = [jnp.exp(s - M) for s in s_ks]
    den = e_list[0]
    for k in range(1, K):
        den = den + e_list[k]
    inv = 1.0 / den
    out = jnp.zeros((C2, TN), jnp.float32)
    for k in range(K):
        w = jnp.sum((e_list[k] * inv)[0:HEADS, :], axis=0, keepdims=True)
        out += hbig[:, k * TN:(k + 1) * TN] * (w * (1.0 / HEADS))
    return out


def _gcn_kernel(G_ref, xt_ref, Wb_ref, bb_ref,
                Wa12_ref, Wb12_ref, bg12_ref, A1_ref, A2_ref,
                A_out_ref, m2a_ref, s1a_ref):
    first = (pl.program_id(0) == 0) & (pl.program_id(1) == 0)

    @pl.when(first)
    def _():
        m2a_ref[...] = jnp.zeros_like(m2a_ref)
        s1a_ref[...] = jnp.zeros_like(s1a_ref)

    xt = xt_ref[0]
    ctr = jnp.maximum(_dot(Wb_ref[...], xt, 1, 0) + bb_ref[...], 0.0)

    g_ks = [G_ref[0, k] for k in range(K)]
    gbig = jnp.concatenate(g_ks, axis=1)

    ubig = _dot(Wa12_ref[...], gbig, 1, 0)
    v12 = _dot(Wb12_ref[...], ctr, 1, 0) + bg12_ref[...]
    vcat = jnp.concatenate([v12] * K, axis=1)
    hbig = _leaky(ubig + vcat)
    out1 = _attn_branch(hbig[:C2], A1_ref[...])
    out2 = _attn_branch(hbig[C2:], A2_ref[...])

    maxg = g_ks[0]
    for k in range(1, K):
        maxg = jnp.maximum(maxg, g_ks[k])

    A_tile = jnp.concatenate([out1, out2, maxg, ctr], axis=0)
    A_out_ref[0] = A_tile
    m2a_ref[...] += _dot(A_tile, A_tile, 1, 1)
    s1a_ref[...] += jnp.sum(A_tile, axis=1, keepdims=True)


def _final_kernel(A_ref, xt_ref, Wd_ref, bd_ref, Wr_ref, br_ref, o_ref):
    A_tile = A_ref[0]
    xt = xt_ref[0]
    dec = jnp.maximum(_dot(Wd_ref[...], A_tile, 1, 0) + bd_ref[...], 0.0)
    res = jnp.maximum(_dot(Wr_ref[...], xt, 1, 0) + br_ref[...], 0.0)
    o_ref[0] = dec + res


def _fold(W, g, bias, S1, M2, count):
    mu = (W @ S1[:, 0]) / count
    e2 = jnp.sum((W @ M2) * W, axis=1) / count
    var = e2 - mu * mu
    s = g / jnp.sqrt(var + EPS)
    return W * s[:, None], (bias - s * mu)[:, None]


def kernel(x, W_res, g_res, b_res, W_btl, g_btl, b_btl, W_g1, g_g1, b_g1, A1,
           W_g2, g_g2, b_g2, A2, W_dec, g_dec, b_dec):
    f32 = jnp.float32
    x = x.astype(f32)
    arb = pltpu.CompilerParams(dimension_semantics=("arbitrary", "arbitrary"))

    m2x, s1x_full = pl.pallas_call(
        _xmom_kernel,
        grid=(B,),
        in_specs=[pl.BlockSpec((1, C, N), lambda b: (b, 0, 0))],
        out_specs=[pl.BlockSpec((C, C), lambda b: (0, 0)),
                   pl.BlockSpec((C, 1), lambda b: (0, 0))],
        out_shape=[jax.ShapeDtypeStruct((C, C), f32),
                   jax.ShapeDtypeStruct((C, 1), f32)],
        compiler_params=pltpu.CompilerParams(dimension_semantics=("arbitrary",)),
    )(x)

    cnt_x = float(B * N)
    Wr, br = _fold(W_res, g_res, b_res, s1x_full, m2x, cnt_x)
    Wb, bb = _fold(W_btl, g_btl, b_btl, s1x_full, m2x, cnt_x)

    G, m2nn, m2cr, m2xx, s1g, s1c = pl.pallas_call(
        _knn_kernel,
        grid=(B, NT),
        in_specs=[pl.BlockSpec((1, C, N), lambda b, i: (b, 0, 0)),
                  pl.BlockSpec((1, C, TN), lambda b, i: (b, 0, i)),
                  pl.BlockSpec((C4, C), lambda b, i: (0, 0)),
                  pl.BlockSpec((C4, 1), lambda b, i: (0, 0))],
        out_specs=[pl.BlockSpec((1, K, C4, TN), lambda b, i: (b, 0, 0, i)),
                   pl.BlockSpec((C4, C4), lambda b, i: (0, 0)),
                   pl.BlockSpec((C4, C4), lambda b, i: (0, 0)),
                   pl.BlockSpec((C4, C4), lambda b, i: (0, 0)),
                   pl.BlockSpec((C4, 1), lambda b, i: (0, 0)),
                   pl.BlockSpec((C4, 1), lambda b, i: (0, 0))],
        out_shape=[jax.ShapeDtypeStruct((B, K, C4, N), f32),
                   jax.ShapeDtypeStruct((C4, C4), f32),
                   jax.ShapeDtypeStruct((C4, C4), f32),
                   jax.ShapeDtypeStruct((C4, C4), f32),
                   jax.ShapeDtypeStruct((C4, 1), f32),
                   jax.ShapeDtypeStruct((C4, 1), f32)],
        compiler_params=arb,
    )(x, x, Wb, bb)

    m2_feat = jnp.concatenate([
        jnp.concatenate([m2nn, m2cr], axis=1),
        jnp.concatenate([m2cr.T, float(K) * m2xx], axis=1)], axis=0)
    s1_feat = jnp.concatenate([s1g, float(K) * s1c], axis=0)
    cnt_e = float(B * N * K)
    Wg1, bg1 = _fold(W_g1, g_g1, b_g1, s1_feat, m2_feat, cnt_e)
    Wg2, bg2 = _fold(W_g2, g_g2, b_g2, s1_feat, m2_feat, cnt_e)

    A1p = jnp.zeros((8, C2), f32).at[:HEADS].set(A1)
    A2p = jnp.zeros((8, C2), f32).at[:HEADS].set(A2)

    wspec = lambda r, c: pl.BlockSpec((r, c), lambda b, i: (0, 0))
    A_feats, m2a, s1a = pl.pallas_call(
        _gcn_kernel,
        grid=(B, NT),
        in_specs=[pl.BlockSpec((1, K, C4, TN), lambda b, i: (b, 0, 0, i)),
                  pl.BlockSpec((1, C, TN), lambda b, i: (b, 0, i)),
                  wspec(C4, C), wspec(C4, 1),
                  wspec(2 * C2, C4), wspec(2 * C2, C4), wspec(2 * C2, 1),
                  wspec(8, C2), wspec(8, C2)],
        out_specs=[pl.BlockSpec((1, 3 * C2, TN), lambda b, i: (b, 0, i)),
                   pl.BlockSpec((3 * C2, 3 * C2), lambda b, i: (0, 0)),
                   pl.BlockSpec((3 * C2, 1), lambda b, i: (0, 0))],
        out_shape=[jax.ShapeDtypeStruct((B, 3 * C2, N), f32),
                   jax.ShapeDtypeStruct((3 * C2, 3 * C2), f32),
                   jax.ShapeDtypeStruct((3 * C2, 1), f32)],
        compiler_params=arb,
    )(G, x, Wb, bb,
      jnp.concatenate([Wg1[:, :C4], Wg2[:, :C4]], axis=0),
      jnp.concatenate([Wg1[:, C4:], Wg2[:, C4:]], axis=0),
      jnp.concatenate([bg1, bg2], axis=0), A1p, A2p)

    Wd, bd = _fold(W_dec, g_dec, b_dec, s1a, m2a, float(B * N))

    out = pl.pallas_call(
        _final_kernel,
        grid=(B, NT),
        in_specs=[pl.BlockSpec((1, 3 * C2, TN), lambda b, i: (b, 0, i)),
                  pl.BlockSpec((1, C, TN), lambda b, i: (b, 0, i)),
                  wspec(OUT, 3 * C2), wspec(OUT, 1),
                  wspec(OUT, C), wspec(OUT, 1)],
        out_specs=[pl.BlockSpec((1, OUT, TN), lambda b, i: (b, 0, i))],
        out_shape=[jax.ShapeDtypeStruct((B, OUT, N), f32)],
        compiler_params=arb,
    )(A_feats, x, Wd, bd, Wr, br)[0]

    return out[..., None]


# speedup vs baseline: 12.9826x; 1.1896x over previous
import functools

import jax
import jax.numpy as jnp
from jax.experimental import pallas as pl
from jax.experimental.pallas import tpu as pltpu

B, C, N, OUT, K, HEADS = 8, 128, 2048, 128, 20, 3
C4, C2 = C // 4, C // 2
TN = 256
NT = N // TN
EPS = 1e-5
HI = jax.lax.Precision.HIGHEST


def _dot(a, b, ca, cb):
    ah = a.astype(jnp.bfloat16)
    al = (a - ah.astype(jnp.float32)).astype(jnp.bfloat16)
    if b is a:
        bh, bl = ah, al
    else:
        bh = b.astype(jnp.bfloat16)
        bl = (b - bh.astype(jnp.float32)).astype(jnp.bfloat16)
    a3 = jnp.concatenate([ah, al, ah], axis=ca)
    b3 = jnp.concatenate([bh, bh, bl], axis=cb)
    return jax.lax.dot_general(
        a3, b3, (((ca,), (cb,)), ((), ())),
        preferred_element_type=jnp.float32)


def _leaky(x, slope=0.2):
    return jnp.where(x >= 0, x, x * slope)


def _xmom_kernel(x_ref, m2_ref, s1_ref):
    @pl.when(pl.program_id(0) == 0)
    def _():
        m2_ref[...] = jnp.zeros_like(m2_ref)
        s1_ref[...] = jnp.zeros_like(s1_ref)

    xb = x_ref[0]
    m2_ref[...] += _dot(xb, xb, 1, 1)
    s1_ref[...] += jnp.sum(xb, axis=1, keepdims=True)


def _knn_kernel(xf_ref, xt_ref, Wb_ref, bb_ref,
                G_ref, m2nn_ref, m2cr_ref, m2xx_ref, s1g_ref, s1x_ref):
    first = (pl.program_id(0) == 0) & (pl.program_id(1) == 0)

    @pl.when(first)
    def _():
        m2nn_ref[...] = jnp.zeros_like(m2nn_ref)
        m2cr_ref[...] = jnp.zeros_like(m2cr_ref)
        m2xx_ref[...] = jnp.zeros_like(m2xx_ref)
        s1g_ref[...] = jnp.zeros_like(s1g_ref)
        s1x_ref[...] = jnp.zeros_like(s1x_ref)

    xf = xf_ref[0]
    xt = xt_ref[0]
    Wb = Wb_ref[...]
    bb = bb_ref[...]

    sq = jnp.sum(xf * xf, axis=0, keepdims=True)
    Xaug = jnp.concatenate([xf * (-2.0), sq], axis=0)
    Yaug = jnp.concatenate([xt, jnp.ones((1, TN), jnp.float32)], axis=0)
    d = _dot(Xaug, Yaug, 0, 0)

    table = jnp.maximum(_dot(Wb, xf, 1, 0) + bb, 0.0)
    ctr = jnp.maximum(_dot(Wb, xt, 1, 0) + bb, 0.0)

    t_hi = table.astype(jnp.bfloat16)
    t_lo = (table - t_hi.astype(jnp.float32)).astype(jnp.bfloat16)
    t_hl = jnp.concatenate([t_hi, t_lo], axis=0)

    iota = jax.lax.broadcasted_iota(jnp.int32, (N, TN), 0)
    gs = []
    for k in range(K):
        am = jnp.argmin(d, axis=0)
        sel = iota == am[None, :]
        sel_bf = jnp.where(sel, 1.0, 0.0).astype(jnp.bfloat16)
        ghl = jax.lax.dot_general(t_hl, sel_bf, (((1,), (0,)), ((), ())),
                                  preferred_element_type=jnp.float32)
        g_k = ghl[:C4] + ghl[C4:]
        G_ref[0, k] = g_k
        gs.append(g_k)
        d = jnp.where(sel, jnp.inf, d)

    gsum = gs[0]
    for k in range(1, K):
        gsum = gsum + gs[k]
    gcat = jnp.concatenate(gs, axis=1)
    m2nn_ref[...] += _dot(gcat, gcat, 1, 1)
    m2cr_ref[...] += _dot(gsum, ctr, 1, 1)
    m2xx_ref[...] += _dot(ctr, ctr, 1, 1)
    s1g_ref[...] += jnp.sum(gsum, axis=1, keepdims=True)
    s1x_ref[...] += jnp.sum(ctr, axis=1, keepdims=True)


def _attn_branch(hbig, Ap):
    sbig = _leaky(_dot(Ap, hbig, 1, 0))
    s_ks = [sbig[:, k * TN:(k + 1) * TN] for k in range(K)]
    M = s_ks[0]
    for k in range(1, K):
        M = jnp.maximum(M, s_ks[k])
    e_list = [jnp.exp(s - M) for s in s_ks]
    den = e_list[0]
    for k in range(1, K):
        den = den + e_list[k]
    inv = 1.0 / den
    out = jnp.zeros((C2, TN), jnp.float32)
    for k in range(K):
        w = jnp.sum((e_list[k] * inv)[0:HEADS, :], axis=0, keepdims=True)
        out += hbig[:, k * TN:(k + 1) * TN] * (w * (1.0 / HEADS))
    return out


def _gcn_kernel(G_ref, xt_ref, Wb_ref, bb_ref,
                Wa12_ref, Wb12_ref, bg12_ref, A1_ref, A2_ref,
                A_out_ref, m2a_ref, s1a_ref):
    first = (pl.program_id(0) == 0) & (pl.program_id(1) == 0)

    @pl.when(first)
    def _():
        m2a_ref[...] = jnp.zeros_like(m2a_ref)
        s1a_ref[...] = jnp.zeros_like(s1a_ref)

    xt = xt_ref[0]
    ctr = jnp.maximum(_dot(Wb_ref[...], xt, 1, 0) + bb_ref[...], 0.0)

    g_ks = [G_ref[0, k] for k in range(K)]
    gbig = jnp.concatenate(g_ks, axis=1)

    ubig = _dot(Wa12_ref[...], gbig, 1, 0)
    v12 = _dot(Wb12_ref[...], ctr, 1, 0) + bg12_ref[...]
    vcat = jnp.concatenate([v12] * K, axis=1)
    hbig = _leaky(ubig + vcat)
    out1 = _attn_branch(hbig[:C2], A1_ref[...])
    out2 = _attn_branch(hbig[C2:], A2_ref[...])

    maxg = g_ks[0]
    for k in range(1, K):
        maxg = jnp.maximum(maxg, g_ks[k])

    A_tile = jnp.concatenate([out1, out2, maxg, ctr], axis=0)
    A_out_ref[0] = A_tile
    m2a_ref[...] += _dot(A_tile, A_tile, 1, 1)
    s1a_ref[...] += jnp.sum(A_tile, axis=1, keepdims=True)


def _final_kernel(A_ref, xt_ref, Wd_ref, bd_ref, Wr_ref, br_ref, o_ref):
    A_tile = A_ref[0]
    xt = xt_ref[0]
    dec = jnp.maximum(_dot(Wd_ref[...], A_tile, 1, 0) + bd_ref[...], 0.0)
    res = jnp.maximum(_dot(Wr_ref[...], xt, 1, 0) + br_ref[...], 0.0)
    o_ref[0] = dec + res


def _fold(W, g, bias, S1, M2, count):
    mu = (W @ S1[:, 0]) / count
    e2 = jnp.sum((W @ M2) * W, axis=1) / count
    var = e2 - mu * mu
    s = g / jnp.sqrt(var + EPS)
    return W * s[:, None], (bias - s * mu)[:, None]


def kernel(x, W_res, g_res, b_res, W_btl, g_btl, b_btl, W_g1, g_g1, b_g1, A1,
           W_g2, g_g2, b_g2, A2, W_dec, g_dec, b_dec):
    f32 = jnp.float32
    x = x.astype(f32)
    arb = pltpu.CompilerParams(dimension_semantics=("arbitrary", "arbitrary"))

    m2x, s1x_full = pl.pallas_call(
        _xmom_kernel,
        grid=(B,),
        in_specs=[pl.BlockSpec((1, C, N), lambda b: (b, 0, 0))],
        out_specs=[pl.BlockSpec((C, C), lambda b: (0, 0)),
                   pl.BlockSpec((C, 1), lambda b: (0, 0))],
        out_shape=[jax.ShapeDtypeStruct((C, C), f32),
                   jax.ShapeDtypeStruct((C, 1), f32)],
        compiler_params=pltpu.CompilerParams(dimension_semantics=("arbitrary",)),
    )(x)

    cnt_x = float(B * N)
    Wr, br = _fold(W_res, g_res, b_res, s1x_full, m2x, cnt_x)
    Wb, bb = _fold(W_btl, g_btl, b_btl, s1x_full, m2x, cnt_x)

    G, m2nn, m2cr, m2xx, s1g, s1c = pl.pallas_call(
        _knn_kernel,
        grid=(B, NT),
        in_specs=[pl.BlockSpec((1, C, N), lambda b, i: (b, 0, 0)),
                  pl.BlockSpec((1, C, TN), lambda b, i: (b, 0, i)),
                  pl.BlockSpec((C4, C), lambda b, i: (0, 0)),
                  pl.BlockSpec((C4, 1), lambda b, i: (0, 0))],
        out_specs=[pl.BlockSpec((1, K, C4, TN), lambda b, i: (b, 0, 0, i)),
                   pl.BlockSpec((C4, C4), lambda b, i: (0, 0)),
                   pl.BlockSpec((C4, C4), lambda b, i: (0, 0)),
                   pl.BlockSpec((C4, C4), lambda b, i: (0, 0)),
                   pl.BlockSpec((C4, 1), lambda b, i: (0, 0)),
                   pl.BlockSpec((C4, 1), lambda b, i: (0, 0))],
        out_shape=[jax.ShapeDtypeStruct((B, K, C4, N), f32),
                   jax.ShapeDtypeStruct((C4, C4), f32),
                   jax.ShapeDtypeStruct((C4, C4), f32),
                   jax.ShapeDtypeStruct((C4, C4), f32),
                   jax.ShapeDtypeStruct((C4, 1), f32),
                   jax.ShapeDtypeStruct((C4, 1), f32)],
        compiler_params=arb,
    )(x, x, Wb, bb)

    m2_feat = jnp.concatenate([
        jnp.concatenate([m2nn, m2cr], axis=1),
        jnp.concatenate([m2cr.T, float(K) * m2xx], axis=1)], axis=0)
    s1_feat = jnp.concatenate([s1g, float(K) * s1c], axis=0)
    cnt_e = float(B * N * K)
    Wg1, bg1 = _fold(W_g1, g_g1, b_g1, s1_feat, m2_feat, cnt_e)
    Wg2, bg2 = _fold(W_g2, g_g2, b_g2, s1_feat, m2_feat, cnt_e)

    A1p = jnp.zeros((8, C2), f32).at[:HEADS].set(A1)
    A2p = jnp.zeros((8, C2), f32).at[:HEADS].set(A2)

    wspec = lambda r, c: pl.BlockSpec((r, c), lambda b, i: (0, 0))
    A_feats, m2a, s1a = pl.pallas_call(
        _gcn_kernel,
        grid=(B, NT),
        in_specs=[pl.BlockSpec((1, K, C4, TN), lambda b, i: (b, 0, 0, i)),
                  pl.BlockSpec((1, C, TN), lambda b, i: (b, 0, i)),
                  wspec(C4, C), wspec(C4, 1),
                  wspec(2 * C2, C4), wspec(2 * C2, C4), wspec(2 * C2, 1),
                  wspec(8, C2), wspec(8, C2)],
        out_specs=[pl.BlockSpec((1, 3 * C2, TN), lambda b, i: (b, 0, i)),
                   pl.BlockSpec((3 * C2, 3 * C2), lambda b, i: (0, 0)),
                   pl.BlockSpec((3 * C2, 1), lambda b, i: (0, 0))],
        out_shape=[jax.ShapeDtypeStruct((B, 3 * C2, N), f32),
                   jax.ShapeDtypeStruct((3 * C2, 3 * C2), f32),
                   jax.ShapeDtypeStruct((3 * C2, 1), f32)],
        compiler_params=arb,
    )(G, x, Wb, bb,
      jnp.concatenate([Wg1[:, :C4], Wg2[:, :C4]], axis=0),
      jnp.concatenate([Wg1[:, C4:], Wg2[:, C4:]], axis=0),
      jnp.concatenate([bg1, bg2], axis=0), A1p, A2p)

    Wd, bd = _fold(W_dec, g_dec, b_dec, s1a, m2a, float(B * N))

    out = pl.pallas_call(
        _final_kernel,
        grid=(B, NT),
        in_specs=[pl.BlockSpec((1, 3 * C2, TN), lambda b, i: (b, 0, i)),
                  pl.BlockSpec((1, C, TN), lambda b, i: (b, 0, i)),
                  wspec(OUT, 3 * C2), wspec(OUT, 1),
                  wspec(OUT, C), wspec(OUT, 1)],
        out_specs=[pl.BlockSpec((1, OUT, TN), lambda b, i: (b, 0, i))],
        out_shape=[jax.ShapeDtypeStruct((B, OUT, N), f32)],
        compiler_params=arb,
    )(A_feats, x, Wd, bd, Wr, br)[0]

    return out[..., None]
